# baseline (speedup 1.0000x reference)
import functools

import jax
import jax.numpy as jnp
from jax import lax
from jax.experimental import pallas as pl
from jax.experimental.pallas import tpu as pltpu
from jax.experimental.pallas import tpu_sc as plsc

N, T, C, H, W = 8, 8, 128, 32, 32
F = C * H * W
TO = 16
NC, NS = 2, 16
NW = NC * NS
WPN = NW // N
CH = 8192
CPW = F // CH // WPN
UNROLL = 8
LANES = 16
NEG = -1e30


@functools.partial(
    pl.kernel,
    out_type=jax.ShapeDtypeStruct((N * TO * F,), jnp.float32),
    mesh=plsc.VectorSubcoreMesh(core_axis_name="c", subcore_axis_name="s"),
    compiler_params=pltpu.CompilerParams(needs_layout_passes=False),
    scratch_types=[
        pltpu.VMEM((T * CH,), jnp.float32),
        pltpu.VMEM((2, CH), jnp.float32),
        pltpu.VMEM((2 * LANES,), jnp.float32),
        pltpu.SemaphoreType.DMA,
        pltpu.SemaphoreType.DMA,
        pltpu.SemaphoreType.DMA,
    ],
)
def _interp_sc(u_hbm, idx_hbm, out_hbm, in_v, out_v, idx_v,
               sem_in, sem_o0, sem_o1):
    cid = lax.axis_index("c")
    sid = lax.axis_index("s")
    wid = sid * NC + cid
    n = wid // WPN
    slot = wid % WPN

    pltpu.sync_copy(idx_hbm.at[pl.ds(n * (2 * LANES), 2 * LANES)], idx_v)
    v1 = idx_v[pl.ds(0, LANES)]
    v2 = idx_v[pl.ds(LANES, LANES)]
    m = jnp.maximum(jnp.max(v1), jnp.max(v2))
    e1 = jnp.exp(v1 - m)
    e2 = jnp.exp(v2 - m)
    tot = jnp.sum(e1) + jnp.sum(e2)
    tf = (plsc.cumsum(e1) * float(T - 1)) / jnp.broadcast_to(tot, (LANES,))
    tl = jnp.minimum(tf.astype(jnp.int32), T - 2)
    alpha = tf - tl.astype(jnp.float32)
    wl = 1.0 - alpha
    wr = alpha

    sems = (sem_o0, sem_o1)
    pending = [None, None]
    for item in range(CPW):
        cbase = (slot * CPW + item) * CH
        copies = []
        for t in range(T):
            cp = pltpu.make_async_copy(
                u_hbm.at[pl.ds((n * T + t) * F + cbase, CH)],
                in_v.at[pl.ds(t * CH, CH)],
                sem_in)
            cp.start()
            copies.append(cp)
        for cp in copies:
            cp.wait()
        for o in range(TO):
            b = (item * TO + o) % 2
            if pending[b] is not None:
                pending[b].wait()
            base_l = tl[o] * CH
            base_r = base_l + CH
            wl_s = wl[o]
            wr_s = wr[o]

            def body(i, carry, base_l=base_l, base_r=base_r, wl_s=wl_s,
                     wr_s=wr_s, b=b):
                for u in range(UNROLL):
                    off = i * (LANES * UNROLL) + u * LANES
                    a = in_v[pl.ds(base_l + off, LANES)]
                    c = in_v[pl.ds(base_r + off, LANES)]
                    out_v[b, pl.ds(off, LANES)] = wl_s * a + wr_s * c
                return carry

            lax.fori_loop(0, CH // (LANES * UNROLL), body, 0)
            cp = pltpu.make_async_copy(
                out_v.at[b],
                out_hbm.at[pl.ds((n * TO + o) * F + cbase, CH)],
                sems[b])
            cp.start()
            pending[b] = cp
    pending[0].wait()
    pending[1].wait()


def kernel(input, index):
    u_flat = input.reshape(-1)
    idx_pad = jnp.pad(index, ((0, 0), (0, 2 * LANES - index.shape[1])),
                      constant_values=NEG).reshape(-1)
    out = _interp_sc(u_flat, idx_pad)
    return out.reshape(N, TO, C, H, W)


# speedup vs baseline: 1.3867x; 1.3867x over previous
import functools

import jax
import jax.numpy as jnp
from jax import lax
from jax.experimental import pallas as pl
from jax.experimental.pallas import tpu as pltpu
from jax.experimental.pallas import tpu_sc as plsc

N, T, C, H, W = 8, 8, 128, 32, 32
F = C * H * W
TO = 16
NC, NS = 2, 16
NW = NC * NS
WPN = NW // N
CH = 8192
CPW = F // CH // WPN
UNROLL = 8
LANES = 16
NEG = -1e30


@functools.partial(
    pl.kernel,
    out_type=jax.ShapeDtypeStruct((N * TO * F,), jnp.float32),
    mesh=plsc.VectorSubcoreMesh(core_axis_name="c", subcore_axis_name="s"),
    compiler_params=pltpu.CompilerParams(needs_layout_passes=False),
    scratch_types=[
        pltpu.VMEM((T * CH,), jnp.float32),
        pltpu.VMEM((2, CH), jnp.float32),
        pltpu.VMEM((2 * LANES,), jnp.float32),
        pltpu.SemaphoreType.DMA,
        pltpu.SemaphoreType.DMA,
        pltpu.SemaphoreType.DMA,
    ],
)
def _interp_sc(u_hbm, idx_hbm, out_hbm, in_v, out_v, idx_v,
               sem_in, sem_o0, sem_o1):
    cid = lax.axis_index("c")
    sid = lax.axis_index("s")
    wid = sid * NC + cid
    n = wid // WPN
    slot = wid % WPN

    pltpu.sync_copy(idx_hbm.at[pl.ds(n * (2 * LANES), 2 * LANES)], idx_v)
    v1 = idx_v[pl.ds(0, LANES)]
    v2 = idx_v[pl.ds(LANES, LANES)]
    m = jnp.maximum(jnp.max(v1), jnp.max(v2))
    e1 = jnp.exp(v1 - m)
    e2 = jnp.exp(v2 - m)
    tot = jnp.sum(e1) + jnp.sum(e2)
    tf = (plsc.cumsum(e1) * float(T - 1)) / jnp.broadcast_to(tot, (LANES,))
    tl = jnp.minimum(tf.astype(jnp.int32), T - 2)
    alpha = tf - tl.astype(jnp.float32)
    wl = 1.0 - alpha
    wr = alpha

    sems = (sem_o0, sem_o1)
    pending = [None, None]
    for item in range(CPW):
        cbase = (slot * CPW + item) * CH
        copies = []
        for t in range(T):
            cp = pltpu.make_async_copy(
                u_hbm.at[pl.ds((n * T + t) * F + cbase, CH)],
                in_v.at[pl.ds(t * CH, CH)],
                sem_in)
            cp.start()
            copies.append(cp)
        for cp in copies:
            cp.wait()
        for o in range(TO):
            b = (item * TO + o) % 2
            if pending[b] is not None:
                pending[b].wait()
            base_l = tl[o] * CH
            base_r = base_l + CH
            wl_s = wl[o]
            wr_s = wr[o]

            def body(off, base_l=base_l, base_r=base_r, wl_s=wl_s,
                     wr_s=wr_s, b=b):
                a = in_v[pl.ds(base_l + off, LANES)]
                c = in_v[pl.ds(base_r + off, LANES)]
                out_v[b, pl.ds(off, LANES)] = wl_s * a + wr_s * c

            plsc.parallel_loop(0, CH, LANES, unroll=UNROLL)(body)
            cp = pltpu.make_async_copy(
                out_v.at[b],
                out_hbm.at[pl.ds((n * TO + o) * F + cbase, CH)],
                sems[b])
            cp.start()
            pending[b] = cp
    pending[0].wait()
    pending[1].wait()


def kernel(input, index):
    u_flat = input.reshape(-1)
    idx_pad = jnp.pad(index, ((0, 0), (0, 2 * LANES - index.shape[1])),
                      constant_values=NEG).reshape(-1)
    out = _interp_sc(u_flat, idx_pad)
    return out.reshape(N, TO, C, H, W)


# speedup vs baseline: 6.0742x; 4.3804x over previous
import functools

import jax
import jax.numpy as jnp
from jax import lax
from jax.experimental import pallas as pl
from jax.experimental.pallas import tpu as pltpu
from jax.experimental.pallas import tpu_sc as plsc

N, T, C, H, W = 8, 8, 128, 32, 32
F = C * H * W
TO = 16
NC, NS = 2, 16
NW = NC * NS
WPN = NW // N
CH = 8192
CPW = F // CH // WPN
UNROLL = 8
LANES = 16
NEG = -1e30


@functools.partial(
    pl.kernel,
    out_type=jax.ShapeDtypeStruct((N * TO * F,), jnp.float32),
    mesh=plsc.VectorSubcoreMesh(core_axis_name="c", subcore_axis_name="s"),
    compiler_params=pltpu.CompilerParams(needs_layout_passes=False),
    scratch_types=[
        pltpu.VMEM((T * CH,), jnp.float32),
        pltpu.VMEM((2, CH), jnp.float32),
        pltpu.VMEM((2 * LANES,), jnp.float32),
        pltpu.SemaphoreType.DMA,
        pltpu.SemaphoreType.DMA,
        pltpu.SemaphoreType.DMA,
    ],
)
def _interp_sc(u_hbm, idx_hbm, out_hbm, in_v, out_v, idx_v,
               sem_in, sem_o0, sem_o1):
    cid = lax.axis_index("c")
    sid = lax.axis_index("s")
    wid = sid * NC + cid
    n = wid // WPN
    slot = wid % WPN

    pltpu.sync_copy(idx_hbm.at[pl.ds(n * (2 * LANES), 2 * LANES)], idx_v)
    v1 = idx_v[pl.ds(0, LANES)]
    v2 = idx_v[pl.ds(LANES, LANES)]
    m = jnp.maximum(jnp.max(v1), jnp.max(v2))
    e1 = jnp.exp(v1 - m)
    e2 = jnp.exp(v2 - m)
    tot = jnp.sum(e1) + jnp.sum(e2)
    tf = (plsc.cumsum(e1) * float(T - 1)) / jnp.broadcast_to(tot, (LANES,))
    tl = jnp.minimum(tf.astype(jnp.int32), T - 2)
    alpha = tf - tl.astype(jnp.float32)
    wl = 1.0 - alpha
    wr = alpha

    sems = (sem_o0, sem_o1)
    pending = [None, None]
    for item in range(CPW):
        cbase = (slot * CPW + item) * CH
        copies = []
        for t in range(T):
            cp = pltpu.make_async_copy(
                u_hbm.at[pl.ds((n * T + t) * F + cbase, CH)],
                in_v.at[pl.ds(t * CH, CH)],
                sem_in)
            cp.start()
            copies.append(cp)
        for cp in copies:
            cp.wait()
        for o in range(TO):
            b = (item * TO + o) % 2
            if pending[b] is not None:
                pending[b].wait()
            base_l = tl[o] * CH
            base_r = base_l + CH
            wl_s = wl[o]
            wr_s = wr[o]

            def body(off, base_l=base_l, base_r=base_r, wl_s=wl_s,
                     wr_s=wr_s, b=b):
                a = in_v[pl.ds(base_l + off, LANES)]
                c = in_v[pl.ds(base_r + off, LANES)]
                out_v[b, pl.ds(off, LANES)] = wl_s * a + wr_s * c

            plsc.parallel_loop(0, CH, LANES, unroll=UNROLL)(body)
            cp = pltpu.make_async_copy(
                out_v.at[b],
                out_hbm.at[pl.ds((n * TO + o) * F + cbase, CH)],
                sems[b])
            cp.start()
            pending[b] = cp
    pending[0].wait()
    pending[1].wait()


def kernel(input, index):
    u_flat = jnp.transpose(input, (0, 1, 3, 4, 2)).reshape(-1)
    idx_pad = jnp.pad(index, ((0, 0), (0, 2 * LANES - index.shape[1])),
                      constant_values=NEG).reshape(-1)
    out = _interp_sc(u_flat, idx_pad)
    return jnp.transpose(out.reshape(N, TO, H, W, C), (0, 1, 4, 2, 3))
